# Initial kernel scaffold; baseline (speedup 1.0000x reference)
import jax
import jax.numpy as jnp
from jax.experimental import pallas as pl


def kernel(logits, labels, ans_emb, print_info):
    raise NotImplementedError("write your pallas kernel here")


# speedup vs baseline: 5.3696x; 5.3696x over previous
import functools

import jax
import jax.numpy as jnp
from jax.experimental import pallas as pl

_ALPHA = 0.5


def _loss_body(logits_ref, labels_ref, emb_ref, out_ref, *, nb):
    step = pl.program_id(0)

    A = emb_ref[...]
    l = logits_ref[...]
    lab = labels_ref[...]
    C = lab.shape[2]

    diff = A - l
    d = jnp.sum(diff * diff, axis=2, keepdims=True)

    m = jnp.max(lab, axis=2, keepdims=True)
    iota_l = jax.lax.broadcasted_iota(jnp.int32, lab.shape, 2)
    is_max = lab == m
    first_idx = jnp.min(jnp.where(is_max, iota_l, C), axis=2, keepdims=True)
    last_idx = jnp.max(jnp.where(is_max, iota_l, -1), axis=2, keepdims=True)

    iota_s = jax.lax.broadcasted_iota(jnp.int32, d.shape, 1)
    d_correct = jnp.sum(
        jnp.where(iota_s == first_idx[:, :, :], d, 0.0), axis=(1, 2))
    hardest = jnp.min(
        jnp.where(iota_s == last_idx[:, :, :], jnp.float32(jnp.inf), d),
        axis=(1, 2))
    loss = jnp.sum(jnp.maximum(d_correct - _ALPHA * hardest, 0.0))

    @pl.when(step == 0)
    def _init():
        out_ref[...] = jnp.zeros_like(out_ref)

    out_ref[...] = out_ref[...] + loss


def kernel(logits, labels, ans_emb, print_info):
    B, C = labels.shape
    D = logits.shape[1]
    nb = 8
    body = functools.partial(_loss_body, nb=nb)
    out = pl.pallas_call(
        body,
        grid=(B // nb,),
        in_specs=[
            pl.BlockSpec((nb, 1, D), lambda i: (i, 0, 0)),
            pl.BlockSpec((nb, 1, C), lambda i: (i, 0, 0)),
            pl.BlockSpec((nb, C, D), lambda i: (i, 0, 0)),
        ],
        out_specs=pl.BlockSpec((1, 1), lambda i: (0, 0)),
        out_shape=jax.ShapeDtypeStruct((1, 1), jnp.float32),
    )(logits.reshape(B, 1, D), labels.reshape(B, 1, C), ans_emb)
    return out[0, 0]


# speedup vs baseline: 6.8970x; 1.2845x over previous
import functools

import jax
import jax.numpy as jnp
from jax.experimental import pallas as pl

_ALPHA = 0.5


def _loss_body(logits_ref, labels_ref, emb_ref, out_ref, *, nb):
    step = pl.program_id(0)

    A = emb_ref[...]
    l = logits_ref[...]
    lab = labels_ref[...]
    C = lab.shape[2]

    G = A * (A - 2.0 * l)
    ones = jnp.ones((l.shape[0], 1, l.shape[2]), jnp.float32)
    dots = jax.lax.dot_general(
        ones, G,
        dimension_numbers=(((2,), (2,)), ((0,), (0,))),
        preferred_element_type=jnp.float32)
    lsq = jnp.sum(l * l, axis=2, keepdims=True)
    d = dots + lsq

    m = jnp.max(lab, axis=2, keepdims=True)
    iota_l = jax.lax.broadcasted_iota(jnp.int32, lab.shape, 2)
    is_max = lab == m
    first_idx = jnp.min(jnp.where(is_max, iota_l, C), axis=2, keepdims=True)
    last_idx = jnp.max(jnp.where(is_max, iota_l, -1), axis=2, keepdims=True)

    d_correct = jnp.sum(
        jnp.where(iota_l == first_idx, d, 0.0), axis=2, keepdims=True)
    hardest = jnp.min(
        jnp.where(iota_l == last_idx, jnp.float32(jnp.inf), d),
        axis=2, keepdims=True)
    loss = jnp.sum(jnp.maximum(d_correct - _ALPHA * hardest, 0.0))

    @pl.when(step == 0)
    def _init():
        out_ref[...] = jnp.zeros_like(out_ref)

    out_ref[...] = out_ref[...] + loss


def kernel(logits, labels, ans_emb, print_info):
    B, C = labels.shape
    D = logits.shape[1]
    nb = 8
    body = functools.partial(_loss_body, nb=nb)
    out = pl.pallas_call(
        body,
        grid=(B // nb,),
        in_specs=[
            pl.BlockSpec((nb, 1, D), lambda i: (i, 0, 0)),
            pl.BlockSpec((nb, 1, C), lambda i: (i, 0, 0)),
            pl.BlockSpec((nb, C, D), lambda i: (i, 0, 0)),
        ],
        out_specs=pl.BlockSpec((1, 1), lambda i: (0, 0)),
        out_shape=jax.ShapeDtypeStruct((1, 1), jnp.float32),
    )(logits.reshape(B, 1, D), labels.reshape(B, 1, C), ans_emb)
    return out[0, 0]


# speedup vs baseline: 6.8994x; 1.0004x over previous
import functools

import jax
import jax.numpy as jnp
from jax.experimental import pallas as pl

_ALPHA = 0.5


def _loss_body(logits_ref, labels_ref, emb_ref, out_ref, *, nb):
    step = pl.program_id(0)

    A = emb_ref[...]
    l = logits_ref[...]
    lab = labels_ref[...]
    C = lab.shape[2]

    G = A * (A - 2.0 * l)
    ones = jnp.ones((l.shape[0], 1, l.shape[2]), jnp.float32)
    dots = jax.lax.dot_general(
        ones, G,
        dimension_numbers=(((2,), (2,)), ((0,), (0,))),
        preferred_element_type=jnp.float32)
    lsq = jnp.sum(l * l, axis=2, keepdims=True)
    d = dots + lsq

    m = jnp.max(lab, axis=2, keepdims=True)
    iota_l = jax.lax.broadcasted_iota(jnp.int32, lab.shape, 2)
    is_max = lab == m
    first_idx = jnp.min(jnp.where(is_max, iota_l, C), axis=2, keepdims=True)
    last_idx = jnp.max(jnp.where(is_max, iota_l, -1), axis=2, keepdims=True)

    d_correct = jnp.sum(
        jnp.where(iota_l == first_idx, d, 0.0), axis=2, keepdims=True)
    hardest = jnp.min(
        jnp.where(iota_l == last_idx, jnp.float32(jnp.inf), d),
        axis=2, keepdims=True)
    loss = jnp.sum(jnp.maximum(d_correct - _ALPHA * hardest, 0.0))

    @pl.when(step == 0)
    def _init():
        out_ref[...] = jnp.zeros_like(out_ref)

    out_ref[...] = out_ref[...] + loss


def kernel(logits, labels, ans_emb, print_info):
    B, C = labels.shape
    D = logits.shape[1]
    nb = 16
    body = functools.partial(_loss_body, nb=nb)
    out = pl.pallas_call(
        body,
        grid=(B // nb,),
        in_specs=[
            pl.BlockSpec((nb, 1, D), lambda i: (i, 0, 0)),
            pl.BlockSpec((nb, 1, C), lambda i: (i, 0, 0)),
            pl.BlockSpec((nb, C, D), lambda i: (i, 0, 0)),
        ],
        out_specs=pl.BlockSpec((1, 1), lambda i: (0, 0)),
        out_shape=jax.ShapeDtypeStruct((1, 1), jnp.float32),
    )(logits.reshape(B, 1, D), labels.reshape(B, 1, C), ans_emb)
    return out[0, 0]
